# Initial kernel scaffold; baseline (speedup 1.0000x reference)
import jax
import jax.numpy as jnp
from jax.experimental import pallas as pl


def kernel(xs, y, similarity, feat_indices, enc_params, dec_params):
    raise NotImplementedError("write your pallas kernel here")


# speedup vs baseline: 1.0034x; 1.0034x over previous
import jax
import jax.numpy as jnp
from jax.experimental import pallas as pl

_B, _K, _C, _HW, _NUM = 8, 3, 128, 64, 32
_N = _K - 1


def _cb(x, p, stride, pad, act='lrelu', bn=True):
    w, b = p[0], p[1]
    x = jnp.pad(x, ((0, 0), (0, 0), (pad, pad), (pad, pad)), mode='reflect')
    x = jax.lax.conv_general_dilated(
        x, w, (stride, stride), 'VALID',
        dimension_numbers=('NCHW', 'OIHW', 'NCHW'))
    x = x + b[None, :, None, None]
    if bn:
        g, be = p[2], p[3]
        m = jnp.mean(x, axis=(0, 2, 3), keepdims=True)
        v = jnp.var(x, axis=(0, 2, 3), keepdims=True)
        x = (x - m) / jnp.sqrt(v + 1e-5)
        x = x * g[None, :, None, None] + be[None, :, None, None]
    if act == 'lrelu':
        x = jnp.where(x >= 0, x, 0.2 * x)
    elif act == 'tanh':
        x = jnp.tanh(x)
    return x


def _fuse_body(feat_ref, refs_ref, wfs_ref, wrefs_ref, idx_ref, sim_ref,
               out_ref, ri_ref):
    feat = feat_ref[0]
    w_fs = wfs_ref[0]
    idx = idx_ref[0]
    p_iota = jax.lax.broadcasted_iota(jnp.int32, (_HW, _NUM), 0)
    onehotT = (idx == p_iota).astype(jnp.float32)
    feat_select = jax.lax.dot_general(
        feat, onehotT, (((1,), (0,)), ((), ())),
        precision=jax.lax.Precision.HIGHEST,
        preferred_element_type=jnp.float32)

    fused = sim_ref[0, 0, 0] * feat_select
    rsum = jnp.zeros_like(feat_select)
    big = jnp.int32(_HW)
    for j in range(_N):
        ref = refs_ref[0, j]
        w_ref = wrefs_ref[0, j]
        fxT = jax.lax.dot_general(
            w_ref, w_fs, (((0,), (0,)), ((), ())),
            preferred_element_type=jnp.float32)
        m = jnp.max(fxT, axis=0, keepdims=True)
        cand = jnp.where(fxT >= m, jax.lax.broadcasted_iota(jnp.int32, (_HW, _NUM), 0), big)
        ind = jnp.min(cand, axis=0, keepdims=True)
        ohT = (ind == p_iota).astype(jnp.float32)
        select = jax.lax.dot_general(
            ref, ohT, (((1,), (0,)), ((), ())),
            precision=jax.lax.Precision.HIGHEST,
            preferred_element_type=jnp.float32)
        rsum = rsum + (sim_ref[0, 0, j + 1].astype(jnp.bfloat16).astype(jnp.float32)
                       * select.astype(jnp.bfloat16).astype(jnp.float32))
        ri_ref[0, 0:1, j * _NUM:(j + 1) * _NUM] = ind
    fused = fused + rsum

    scat = jax.lax.dot_general(
        fused, onehotT, (((1,), (1,)), ((), ())),
        precision=jax.lax.Precision.HIGHEST,
        preferred_element_type=jnp.float32)
    cnt = jax.lax.dot_general(
        jnp.ones((_C, _NUM), jnp.float32), onehotT, (((1,), (1,)), ((), ())),
        precision=jax.lax.Precision.HIGHEST,
        preferred_element_type=jnp.float32)
    out_ref[0] = jnp.where(cnt > 0, scat / jnp.maximum(cnt, 1.0), feat)


def _nrm(x, axis):
    n = jnp.linalg.norm(x, axis=axis, keepdims=True)
    return x / jnp.maximum(n, 1e-12)


def _fuse(feat, refs, feat_indices, sim):
    w_feat = _nrm(jnp.transpose(feat, (0, 2, 1)), 2)
    w_refs = _nrm(jnp.transpose(
        refs.reshape(_B, _N, _C, _HW), (0, 2, 1, 3)).reshape(_B, _C, _N * _HW), 1)
    w_fs = jnp.take_along_axis(w_feat, feat_indices[:, :, None], axis=1)
    w_fs = _nrm(w_fs, 2)
    w_refs4 = w_refs.reshape(_B, _C, _N, _HW)
    wfs_in = jnp.transpose(w_fs, (0, 2, 1)).astype(jnp.bfloat16)
    wrefs_in = jnp.transpose(w_refs4, (0, 2, 1, 3)).astype(jnp.bfloat16)

    idx3 = feat_indices.reshape(_B, 1, _NUM).astype(jnp.int32)
    sim3 = sim.reshape(_B, 1, _K)
    out, ri = pl.pallas_call(
        _fuse_body,
        grid=(_B,),
        in_specs=[
            pl.BlockSpec((1, _C, _HW), lambda i: (i, 0, 0)),
            pl.BlockSpec((1, _N, _C, _HW), lambda i: (i, 0, 0, 0)),
            pl.BlockSpec((1, _C, _NUM), lambda i: (i, 0, 0)),
            pl.BlockSpec((1, _N, _C, _HW), lambda i: (i, 0, 0, 0)),
            pl.BlockSpec((1, 1, _NUM), lambda i: (i, 0, 0)),
            pl.BlockSpec((1, 1, _K), lambda i: (i, 0, 0)),
        ],
        out_specs=[
            pl.BlockSpec((1, _C, _HW), lambda i: (i, 0, 0)),
            pl.BlockSpec((1, 1, _N * _NUM), lambda i: (i, 0, 0)),
        ],
        out_shape=[
            jax.ShapeDtypeStruct((_B, _C, _HW), jnp.float32),
            jax.ShapeDtypeStruct((_B, 1, _N * _NUM), jnp.int32),
        ],
    )(feat, refs.reshape(_B, _N, _C, _HW), wfs_in, wrefs_in, idx3, sim3)
    return out, ri.reshape(_B, _N, _NUM)


def kernel(xs, y, similarity, feat_indices, enc_params, dec_params):
    b, k, C, H, W = xs.shape
    x = xs.reshape(b * k, C, H, W)
    strides = [1, 2, 2, 2, 2]
    pads = [2, 1, 1, 1, 1]
    for p, s, pd in zip(enc_params, strides, pads):
        x = _cb(x, p, s, pd, act='lrelu', bn=True)
    c, h, w = x.shape[-3:]
    querys = x.reshape(b, k, c, h * w)
    sim = similarity / jnp.sum(similarity, axis=1, keepdims=True)
    feat = querys[:, 0]
    refs = querys[:, 1:]

    feat_out, ri = _fuse(feat, refs, feat_indices, sim)

    z = feat_out.reshape(b, c, h, w)
    dec_bn = [True, True, True, True, False]
    dec_act = ['lrelu', 'lrelu', 'lrelu', 'lrelu', 'tanh']
    dec_pad = [1, 1, 1, 1, 2]
    for i2, p in enumerate(dec_params):
        if i2 < 4:
            z = jnp.repeat(jnp.repeat(z, 2, axis=2), 2, axis=3)
        z = _cb(z, p, 1, dec_pad[i2], act=dec_act[i2], bn=dec_bn[i2])
    return (z, sim, feat_indices, ri)
